# Initial kernel scaffold; baseline (speedup 1.0000x reference)
import jax
import jax.numpy as jnp
from jax.experimental import pallas as pl


def kernel(features, node, neighbours, raw_features, neigh_weights, i):
    raise NotImplementedError("write your pallas kernel here")


# speedup vs baseline: 8.6797x; 8.6797x over previous
import functools

import jax
import jax.numpy as jnp
from jax import lax
from jax.experimental import pallas as pl
from jax.experimental.pallas import tpu as pltpu
from jax.experimental.pallas import tpu_sc as plsc

N_NODES = 100000
D = 128
B = 16384
K = 32
NC = 2
NS = 16
NW = NC * NS
BPW = B // NW
SC_ELEMS = 128
N_SC = BPW // SC_ELEMS
GROUP = 4
N_GROUPS = SC_ELEMS // GROUP
ROWS = GROUP * K
NV = D // 16


def _sc_body(table, nidx_hbm, node_hbm, out_hbm,
             nidx_v, node_v, node_buf, out_buf, nbuf0, nbuf1,
             sem_n, sem0, sem1):
  wid = lax.axis_index("s") * NC + lax.axis_index("c")
  base = wid * BPW

  pltpu.sync_copy(nidx_hbm.at[pl.ds(base * K, BPW * K)], nidx_v)
  pltpu.sync_copy(node_hbm.at[pl.ds(base, BPW)], node_v)

  nbufs = (nbuf0, nbuf1)
  sems = (sem0, sem1)

  def superchunk(sc_i, _):
    pltpu.async_copy(
        table.at[node_v.at[pl.ds(sc_i * SC_ELEMS, SC_ELEMS)]],
        node_buf, sem_n).wait()

    def issue(g, buf, sem):
      off = (sc_i * N_GROUPS + g) * ROWS
      pltpu.async_copy(table.at[nidx_v.at[pl.ds(off, ROWS)]], buf, sem)

    def wait(g, buf, sem):
      off = (sc_i * N_GROUPS + g) * ROWS
      pltpu.make_async_copy(
          table.at[nidx_v.at[pl.ds(off, ROWS)]], buf, sem).wait()

    issue(0, nbufs[0], sems[0])

    def pair(i, _):
      for b2 in range(2):
        g = 2 * i + b2
        buf, sem = nbufs[b2], sems[b2]

        @pl.when(g < N_GROUPS - 1)
        def _():
          issue(g + 1, nbufs[1 - b2], sems[1 - b2])

        wait(g, buf, sem)
        for el in range(GROUP):
          row = g * GROUP + el

          def red(r, acc):
            return tuple(acc[j] + buf[el * K + r, pl.ds(16 * j, 16)]
                         for j in range(NV))

          acc = tuple(node_buf[row, pl.ds(16 * j, 16)] for j in range(NV))
          acc = lax.fori_loop(0, K, red, acc)
          for j in range(NV):
            out_buf[row, pl.ds(16 * j, 16)] = acc[j]
      return ()

    lax.fori_loop(0, N_GROUPS // 2, pair, ())
    pltpu.sync_copy(out_buf, out_hbm.at[pl.ds(base + sc_i * SC_ELEMS, SC_ELEMS)])
    return ()

  lax.fori_loop(0, N_SC, superchunk, ())


_gather_sum = functools.partial(
    pl.kernel,
    out_type=jax.ShapeDtypeStruct((B, D), jnp.float32),
    mesh=plsc.VectorSubcoreMesh(core_axis_name="c", subcore_axis_name="s"),
    scratch_types=[
        pltpu.VMEM((BPW * K,), jnp.int32),
        pltpu.VMEM((BPW,), jnp.int32),
        pltpu.VMEM((SC_ELEMS, D), jnp.float32),
        pltpu.VMEM((SC_ELEMS, D), jnp.float32),
        pltpu.VMEM((ROWS, D), jnp.float32),
        pltpu.VMEM((ROWS, D), jnp.float32),
        pltpu.SemaphoreType.DMA,
        pltpu.SemaphoreType.DMA,
        pltpu.SemaphoreType.DMA,
    ],
)(_sc_body)


def _mm_body(x_ref, w_ref, o_ref):
  y = jnp.dot(x_ref[...], w_ref[...], preferred_element_type=jnp.float32)
  o_ref[...] = jnp.maximum(y, 0.0) * (1.0 / (K + 1))


BM = 2048

_matmul_relu = pl.pallas_call(
    _mm_body,
    grid=(B // BM,),
    in_specs=[
        pl.BlockSpec((BM, D), lambda m: (m, 0)),
        pl.BlockSpec((D, D), lambda m: (0, 0)),
    ],
    out_specs=pl.BlockSpec((BM, D), lambda m: (m, 0)),
    out_shape=jax.ShapeDtypeStruct((B, D), jnp.float32),
)


@jax.jit
def kernel(features, node, neighbours, raw_features, neigh_weights, i):
  nidx = neighbours.astype(jnp.int32).reshape(-1)
  node_idx = node.astype(jnp.int32).reshape(-1)
  sums = _gather_sum(features, nidx, node_idx)
  out = _matmul_relu(sums, neigh_weights)
  return (out, raw_features)
